# baseline (device time: 182246 ns/iter reference)
import jax
import jax.numpy as jnp
from jax import lax
from jax.experimental import pallas as pl
from jax.experimental.pallas import tpu as pltpu

N_DEV = 16
M_PER = 256
N_SUB = 256


def kernel(x, w_mat):
    m_tot, k_per = x.shape
    _, n = w_mat.shape
    nh = n // 2
    n_sub = nh // N_SUB
    n_rings = 2 * n_sub

    def body(x_ref, w_ref, out_ref, *scratch):
        my = lax.axis_index("i")
        left = lax.rem(my + N_DEV - 1, N_DEV)
        right = lax.rem(my + 1, N_DEV)

        rings = []
        for q in range(n_rings):
            is_right = q % 2 == 0
            h = q // 2
            rings.append(dict(
                send_buf=scratch[q],
                comm_buf=scratch[n_rings + q],
                send_sems=scratch[2 * n_rings + q],
                recv_sems=scratch[3 * n_rings + q],
                credit=scratch[4 * n_rings + q],
                dst=right if is_right else left,
                upstream=left if is_right else right,
                is_right=is_right,
                col0=(0 if is_right else nh) + h * N_SUB,
                rdmas=[],
            ))

        barrier_sem = pltpu.get_barrier_semaphore()
        for nbr in (left, right):
            pl.semaphore_signal(
                barrier_sem, inc=1,
                device_id=(nbr,), device_id_type=pl.DeviceIdType.MESH,
            )

        def partial(c, lo, hi):
            rows = x_ref[pl.ds(c * M_PER, M_PER), :]
            return jnp.dot(rows, w_ref[:, lo:hi],
                           preferred_element_type=jnp.float32)

        def start_hop(ring, s):
            slot = s % 2
            if s >= 2:
                pl.semaphore_wait(ring["credit"], 1)
            rdma = pltpu.make_async_remote_copy(
                src_ref=ring["send_buf"].at[slot],
                dst_ref=ring["comm_buf"].at[slot],
                send_sem=ring["send_sems"].at[slot],
                recv_sem=ring["recv_sems"].at[slot],
                device_id=(ring["dst"],),
                device_id_type=pl.DeviceIdType.MESH,
            )
            rdma.start()
            ring["rdmas"].append(rdma)

        c_seed_r = lax.rem(my + N_DEV - 1, N_DEV)
        c_seed_l = lax.rem(my + 1, N_DEV)
        for h in range(n_sub):
            for ring in rings[2 * h:2 * h + 2]:
                c = c_seed_r if ring["is_right"] else c_seed_l
                ring["send_buf"][0, :, :] = partial(
                    c, ring["col0"], ring["col0"] + N_SUB
                )
            if h == 0:
                pl.semaphore_wait(barrier_sem, 2)
            for ring in rings[2 * h:2 * h + 2]:
                start_hop(ring, 0)

        for s in range(N_DEV - 1):
            slot = s % 2
            nxt_r = partial(lax.rem(my + (N_DEV - 2 - s), N_DEV), 0, nh)
            nxt_l = partial(lax.rem(my + 2 + s, N_DEV), nh, n)

            for ring in rings:
                nxt = nxt_r if ring["is_right"] else nxt_l
                lo = ring["col0"] - (0 if ring["is_right"] else nh)
                sub = nxt[:, lo:lo + N_SUB]

                ring["rdmas"][s].wait_recv()
                if s < N_DEV - 2:
                    if s >= 1:
                        ring["rdmas"][s - 1].wait_send()
                    ring["send_buf"][(s + 1) % 2, :, :] = (
                        ring["comm_buf"][slot, :, :] + sub
                    )
                    start_hop(ring, s + 1)
                else:
                    c0 = ring["col0"]
                    out_ref[:, c0:c0 + N_SUB] = (
                        ring["comm_buf"][slot, :, :] + sub
                    )
                if s <= N_DEV - 4:
                    pl.semaphore_signal(
                        ring["credit"], inc=1,
                        device_id=(ring["upstream"],),
                        device_id_type=pl.DeviceIdType.MESH,
                    )

        for ring in rings:
            ring["rdmas"][N_DEV - 3].wait_send()
            ring["rdmas"][N_DEV - 2].wait_send()

    return pl.pallas_call(
        body,
        out_shape=jax.ShapeDtypeStruct((M_PER, n), jnp.float32),
        in_specs=[
            pl.BlockSpec(memory_space=pltpu.VMEM),
            pl.BlockSpec(memory_space=pltpu.VMEM),
        ],
        out_specs=pl.BlockSpec(memory_space=pltpu.VMEM),
        scratch_shapes=(
            [pltpu.VMEM((2, M_PER, N_SUB), jnp.float32)] * n_rings
            + [pltpu.VMEM((2, M_PER, N_SUB), jnp.float32)] * n_rings
            + [pltpu.SemaphoreType.DMA((2,))] * n_rings
            + [pltpu.SemaphoreType.DMA((2,))] * n_rings
            + [pltpu.SemaphoreType.REGULAR] * n_rings
        ),
        compiler_params=pltpu.CompilerParams(collective_id=0),
    )(x, w_mat)


# device time: 181803 ns/iter; 1.0024x vs baseline; 1.0024x over previous
import jax
import jax.numpy as jnp
from jax import lax
from jax.experimental import pallas as pl
from jax.experimental.pallas import tpu as pltpu

N_DEV = 16
M_PER = 256
N_SUB = 512


def kernel(x, w_mat):
    m_tot, k_per = x.shape
    _, n = w_mat.shape
    nh = n // 2
    n_sub = nh // N_SUB
    n_rings = 2 * n_sub

    def body(x_ref, w_ref, out_ref, *scratch):
        my = lax.axis_index("i")
        left = lax.rem(my + N_DEV - 1, N_DEV)
        right = lax.rem(my + 1, N_DEV)

        rings = []
        for q in range(n_rings):
            is_right = q % 2 == 0
            h = q // 2
            rings.append(dict(
                send_buf=scratch[q],
                comm_buf=scratch[n_rings + q],
                send_sems=scratch[2 * n_rings + q],
                recv_sems=scratch[3 * n_rings + q],
                credit=scratch[4 * n_rings + q],
                dst=right if is_right else left,
                upstream=left if is_right else right,
                is_right=is_right,
                col0=(0 if is_right else nh) + h * N_SUB,
                rdmas=[],
            ))

        barrier_sem = pltpu.get_barrier_semaphore()
        for nbr in (left, right):
            pl.semaphore_signal(
                barrier_sem, inc=1,
                device_id=(nbr,), device_id_type=pl.DeviceIdType.MESH,
            )

        def partial(c, lo, hi):
            rows = x_ref[pl.ds(c * M_PER, M_PER), :]
            return jnp.dot(rows, w_ref[:, lo:hi],
                           preferred_element_type=jnp.float32)

        def start_hop(ring, s):
            slot = s % 2
            if s >= 2:
                pl.semaphore_wait(ring["credit"], 1)
            rdma = pltpu.make_async_remote_copy(
                src_ref=ring["send_buf"].at[slot],
                dst_ref=ring["comm_buf"].at[slot],
                send_sem=ring["send_sems"].at[slot],
                recv_sem=ring["recv_sems"].at[slot],
                device_id=(ring["dst"],),
                device_id_type=pl.DeviceIdType.MESH,
            )
            rdma.start()
            ring["rdmas"].append(rdma)

        c_seed_r = lax.rem(my + N_DEV - 1, N_DEV)
        c_seed_l = lax.rem(my + 1, N_DEV)
        for h in range(n_sub):
            for ring in rings[2 * h:2 * h + 2]:
                c = c_seed_r if ring["is_right"] else c_seed_l
                ring["send_buf"][0, :, :] = partial(
                    c, ring["col0"], ring["col0"] + N_SUB
                )
            if h == 0:
                pl.semaphore_wait(barrier_sem, 2)
            for ring in rings[2 * h:2 * h + 2]:
                start_hop(ring, 0)

        for s in range(N_DEV - 1):
            slot = s % 2
            nxt_r = partial(lax.rem(my + (N_DEV - 2 - s), N_DEV), 0, nh)
            nxt_l = partial(lax.rem(my + 2 + s, N_DEV), nh, n)

            for ring in rings:
                nxt = nxt_r if ring["is_right"] else nxt_l
                lo = ring["col0"] - (0 if ring["is_right"] else nh)
                sub = nxt[:, lo:lo + N_SUB]

                ring["rdmas"][s].wait_recv()
                if s < N_DEV - 2:
                    if s >= 1:
                        ring["rdmas"][s - 1].wait_send()
                    ring["send_buf"][(s + 1) % 2, :, :] = (
                        ring["comm_buf"][slot, :, :] + sub
                    )
                    start_hop(ring, s + 1)
                else:
                    c0 = ring["col0"]
                    out_ref[:, c0:c0 + N_SUB] = (
                        ring["comm_buf"][slot, :, :] + sub
                    )
                if s <= N_DEV - 4:
                    pl.semaphore_signal(
                        ring["credit"], inc=1,
                        device_id=(ring["upstream"],),
                        device_id_type=pl.DeviceIdType.MESH,
                    )

        for ring in rings:
            ring["rdmas"][N_DEV - 3].wait_send()
            ring["rdmas"][N_DEV - 2].wait_send()

    return pl.pallas_call(
        body,
        out_shape=jax.ShapeDtypeStruct((M_PER, n), jnp.float32),
        in_specs=[
            pl.BlockSpec(memory_space=pltpu.VMEM),
            pl.BlockSpec(memory_space=pltpu.VMEM),
        ],
        out_specs=pl.BlockSpec(memory_space=pltpu.VMEM),
        scratch_shapes=(
            [pltpu.VMEM((2, M_PER, N_SUB), jnp.float32)] * n_rings
            + [pltpu.VMEM((2, M_PER, N_SUB), jnp.float32)] * n_rings
            + [pltpu.SemaphoreType.DMA((2,))] * n_rings
            + [pltpu.SemaphoreType.DMA((2,))] * n_rings
            + [pltpu.SemaphoreType.REGULAR] * n_rings
        ),
        compiler_params=pltpu.CompilerParams(collective_id=0),
    )(x, w_mat)
